# Initial kernel scaffold; baseline (speedup 1.0000x reference)
import jax
import jax.numpy as jnp
from jax.experimental import pallas as pl


def kernel(h, edge_index, Wm1, bm1, Wm2, bm2, Wa, ba, Wh1, bh1, Wh2, bh2, Wo1, bo1, Wo2, bo2, Wo3, bo3):
    raise NotImplementedError("write your pallas kernel here")


# speedup vs baseline: 2.9841x; 2.9841x over previous
import functools

import jax
import jax.numpy as jnp
from jax import lax
from jax.experimental import pallas as pl
from jax.experimental.pallas import tpu as pltpu
from jax.experimental.pallas import tpu_sc as plsc

_N = 10000
_E = 160000
_D = 256
_HID = 256
_OUT = 128
_L = 4

_NC = 2
_NS = 16
_NW = _NC * _NS
_LANE = 16

_EW = _E // _NW
_GC = 64
_GT = 80

_SE = _E // _NS
_SCC = 80
_SCT = _SE // _SCC
_FH = _HID // _NC
_NR = 640
_NPAD = _NR * _NS
_NTAIL = _N - 15 * _NR


def _silu(x):
    return x * jax.nn.sigmoid(x)


def _tc_proj(h, w1a, w1b, bm1):
    bn = 2000

    def body(h_ref, wa_ref, wb_ref, b_ref, a_ref, bo_ref):
        hb = h_ref[...]
        a_ref[...] = jnp.dot(hb, wa_ref[...],
                             preferred_element_type=jnp.float32) + b_ref[...]
        bo_ref[...] = jnp.dot(hb, wb_ref[...],
                              preferred_element_type=jnp.float32)

    return pl.pallas_call(
        body,
        grid=(_N // bn,),
        in_specs=[
            pl.BlockSpec((bn, _D), lambda i: (i, 0)),
            pl.BlockSpec((_D, _HID), lambda i: (0, 0)),
            pl.BlockSpec((_D, _HID), lambda i: (0, 0)),
            pl.BlockSpec((1, _HID), lambda i: (0, 0)),
        ],
        out_specs=[pl.BlockSpec((bn, _HID), lambda i: (i, 0))] * 2,
        out_shape=[jax.ShapeDtypeStruct((_N, _HID), jnp.float32)] * 2,
    )(h, w1a, w1b, bm1.reshape(1, _HID))


def _tc_edge(g, wm2, bm2, wa_row, ba):
    be = 1600

    def body(g_ref, w_ref, b_ref, wa_ref, ba_ref, o_ref):
        m = _silu(g_ref[...])
        m2 = _silu(jnp.dot(m, w_ref[...],
                           preferred_element_type=jnp.float32) + b_ref[...])
        logit = jnp.sum(m2 * wa_ref[...], axis=1, keepdims=True) + ba_ref[0, 0]
        o_ref[...] = m2 * jax.nn.sigmoid(logit)

    return pl.pallas_call(
        body,
        grid=(_E // be,),
        in_specs=[
            pl.BlockSpec((be, _HID), lambda i: (i, 0)),
            pl.BlockSpec((_HID, _HID), lambda i: (0, 0)),
            pl.BlockSpec((1, _HID), lambda i: (0, 0)),
            pl.BlockSpec((1, _HID), lambda i: (0, 0)),
            pl.BlockSpec((1, 1), lambda i: (0, 0)),
        ],
        out_specs=pl.BlockSpec((be, _HID), lambda i: (i, 0)),
        out_shape=jax.ShapeDtypeStruct((_E, _HID), jnp.float32),
    )(g, wm2, bm2.reshape(1, _HID), wa_row, ba.reshape(1, 1))


def _tc_node(h, agg, cnt2, wh1h, wh1m, bh1, wh2, bh2):
    bn = 2000

    def body(h_ref, agg_ref, c_ref, w1h_ref, w1m_ref, b1_ref, w2_ref, b2_ref,
             o_ref):
        denom = jnp.maximum(c_ref[:, 0:1], 1.0)
        mean = agg_ref[...] / denom
        z = _silu(
            jnp.dot(h_ref[...], w1h_ref[...],
                    preferred_element_type=jnp.float32)
            + jnp.dot(mean, w1m_ref[...], preferred_element_type=jnp.float32)
            + b1_ref[...])
        o_ref[...] = jnp.dot(z, w2_ref[...],
                             preferred_element_type=jnp.float32) + b2_ref[...]

    return pl.pallas_call(
        body,
        grid=(_N // bn,),
        in_specs=[
            pl.BlockSpec((bn, _D), lambda i: (i, 0)),
            pl.BlockSpec((bn, _HID), lambda i: (i, 0)),
            pl.BlockSpec((bn, _FH), lambda i: (i, 0)),
            pl.BlockSpec((_D, _HID), lambda i: (0, 0)),
            pl.BlockSpec((_HID, _HID), lambda i: (0, 0)),
            pl.BlockSpec((1, _HID), lambda i: (0, 0)),
            pl.BlockSpec((_HID, _HID), lambda i: (0, 0)),
            pl.BlockSpec((1, _HID), lambda i: (0, 0)),
        ],
        out_specs=pl.BlockSpec((bn, _HID), lambda i: (i, 0)),
        out_shape=jax.ShapeDtypeStruct((_N, _HID), jnp.float32),
    )(h, agg, cnt2, wh1h, wh1m, bh1.reshape(1, _HID), wh2,
      bh2.reshape(1, _HID))


def _tc_final(h, wo1, bo1, wo2, bo2, wo3, bo3):
    bn = 1000

    def body(h_ref, w1_ref, b1_ref, w2_ref, b2_ref, w3_ref, b3_ref, o_ref):
        t = _silu(jnp.dot(h_ref[...], w1_ref[...],
                          preferred_element_type=jnp.float32) + b1_ref[...])
        t = jax.nn.relu(jnp.dot(t, w2_ref[...],
                                preferred_element_type=jnp.float32)
                        + b2_ref[...])
        o_ref[...] = jnp.dot(t, w3_ref[...],
                             preferred_element_type=jnp.float32) + b3_ref[...]

    return pl.pallas_call(
        body,
        grid=(_N // bn,),
        in_specs=[
            pl.BlockSpec((bn, _HID), lambda i: (i, 0)),
            pl.BlockSpec((_HID, 1024), lambda i: (0, 0)),
            pl.BlockSpec((1, 1024), lambda i: (0, 0)),
            pl.BlockSpec((1024, 1024), lambda i: (0, 0)),
            pl.BlockSpec((1, 1024), lambda i: (0, 0)),
            pl.BlockSpec((1024, _OUT), lambda i: (0, 0)),
            pl.BlockSpec((1, _OUT), lambda i: (0, 0)),
        ],
        out_specs=pl.BlockSpec((bn, _OUT), lambda i: (i, 0)),
        out_shape=jax.ShapeDtypeStruct((_N, _OUT), jnp.float32),
    )(h, wo1, bo1.reshape(1, 1024), wo2, bo2.reshape(1, 1024), wo3,
      bo3.reshape(1, _OUT))


def _ring(n_chunks, start, process):
    start(0, 0)
    start(1, 1)

    def pair(p, carry):
        for s in (0, 1):
            t = 2 * p + s

            @pl.when(t < n_chunks)
            def _():
                process(t, s)

                @pl.when(t + 2 < n_chunks)
                def _():
                    start(t + 2, s)

        return carry

    lax.fori_loop(0, (n_chunks + 1) // 2, pair, 0)


def _sc_gather(a, b, idx_i, idx_j):
    mesh = plsc.VectorSubcoreMesh(core_axis_name="c", subcore_axis_name="s")

    @functools.partial(
        pl.kernel,
        out_type=jax.ShapeDtypeStruct((_E, _HID), jnp.float32),
        mesh=mesh,
        scratch_types=[
            pltpu.VMEM((2, _GC), jnp.int32),
            pltpu.VMEM((2, _GC), jnp.int32),
            pltpu.VMEM((2, _GC, _HID), jnp.float32),
            pltpu.VMEM((2, _GC, _HID), jnp.float32),
            pltpu.SemaphoreType.DMA,
            pltpu.SemaphoreType.DMA,
            pltpu.SemaphoreType.DMA,
            pltpu.SemaphoreType.DMA,
        ],
    )
    def k(a_hbm, b_hbm, ii_hbm, jj_hbm, g_hbm, ii_v, jj_v, ra_v, rb_v,
          gsem0, gsem1, osem0, osem1):
        cid = lax.axis_index("c")
        sid = lax.axis_index("s")
        wid = sid * _NC + cid
        base = wid * _EW
        gsems = (gsem0, gsem1)
        osems = (osem0, osem1)

        def off_of(t):
            return base + jnp.minimum(t * _GC, _EW - _GC)

        def start(t, s):
            off = off_of(t)
            pltpu.sync_copy(ii_hbm.at[pl.ds(off, _GC)], ii_v.at[s])
            pltpu.sync_copy(jj_hbm.at[pl.ds(off, _GC)], jj_v.at[s])
            pltpu.async_copy(a_hbm.at[ii_v.at[s]], ra_v.at[s], gsems[s])
            pltpu.async_copy(b_hbm.at[jj_v.at[s]], rb_v.at[s], gsems[s])

        def process(t, s):
            pltpu.make_async_copy(a_hbm.at[ii_v.at[s]], ra_v.at[s],
                                  gsems[s]).wait()
            pltpu.make_async_copy(b_hbm.at[jj_v.at[s]], rb_v.at[s],
                                  gsems[s]).wait()

            def addrow(e, carry):
                for kk in range(_HID // _LANE):
                    sl = pl.ds(kk * _LANE, _LANE)
                    ra_v[s, e, sl] = ra_v[s, e, sl] + rb_v[s, e, sl]
                return carry

            lax.fori_loop(0, _GC, addrow, 0)
            out = pltpu.async_copy(ra_v.at[s],
                                   g_hbm.at[pl.ds(off_of(t), _GC)], osems[s])
            out.wait()

        _ring(_GT, start, process)

    return k(a, b, idx_i, idx_j)


def _zero_block(z_v, rows):
    def zrow(r, carry):
        zero = jnp.zeros((_LANE,), jnp.float32)
        for kk in range(_FH // _LANE):
            z_v[r, pl.ds(kk * _LANE, _LANE)] = zero
        return carry

    lax.fori_loop(0, rows, zrow, 0)


def _sc_scatter(msg, idx_i):
    mesh = plsc.VectorSubcoreMesh(core_axis_name="c", subcore_axis_name="s")

    @functools.partial(
        pl.kernel,
        out_type=jax.ShapeDtypeStruct((_N, _HID), jnp.float32),
        mesh=mesh,
        scratch_types=[
            pltpu.VMEM((2, _SCC), jnp.int32),
            pltpu.VMEM((2, _SCC, _FH), jnp.float32),
            pltpu.VMEM((_SCC, _FH), jnp.float32),
            pltpu.VMEM_SHARED((_NPAD, _FH), jnp.float32),
            pltpu.SemaphoreType.DMA,
            pltpu.SemaphoreType.DMA,
        ],
    )
    def k(msg_hbm, ii_hbm, agg_hbm, ii_v, mb_v, z_v, acc_sh, sem0, sem1):
        cid = lax.axis_index("c")
        sid = lax.axis_index("s")
        base = sid * _SE
        col0 = cid * _FH
        sems = (sem0, sem1)
        row0 = sid * _NR

        _zero_block(z_v, _SCC)
        for zc in range(_NR // _SCC):
            pltpu.sync_copy(z_v, acc_sh.at[pl.ds(row0 + zc * _SCC, _SCC)])
        plsc.subcore_barrier()

        def start(t, s):
            off = base + t * _SCC
            pltpu.sync_copy(ii_hbm.at[pl.ds(off, _SCC)], ii_v.at[s])
            pltpu.async_copy(
                msg_hbm.at[pl.ds(off, _SCC), pl.ds(col0, _FH)], mb_v.at[s],
                sems[s])

        def process(t, s):
            pltpu.make_async_copy(
                msg_hbm.at[pl.ds(base + t * _SCC, _SCC), pl.ds(col0, _FH)],
                mb_v.at[s], sems[s]).wait()
            pltpu.sync_copy(mb_v.at[s], acc_sh.at[ii_v.at[s]], add=True)

        _ring(_SCT, start, process)
        plsc.subcore_barrier()

        @pl.when(sid < _NS - 1)
        def _():
            pltpu.sync_copy(acc_sh.at[pl.ds(row0, _NR)],
                            agg_hbm.at[pl.ds(row0, _NR), pl.ds(col0, _FH)])

        @pl.when(sid == _NS - 1)
        def _():
            pltpu.sync_copy(
                acc_sh.at[pl.ds(row0, _NTAIL)],
                agg_hbm.at[pl.ds(row0, _NTAIL), pl.ds(col0, _FH)])

    return k(msg, idx_i)


def kernel(h, edge_index, Wm1, bm1, Wm2, bm2, Wa, ba, Wh1, bh1, Wh2, bh2,
           Wo1, bo1, Wo2, bo2, Wo3, bo3):
    idx_i = edge_index[0]
    idx_j = edge_index[1]
    cnt2 = _sc_scatter(jnp.ones((_E, _HID), jnp.float32), idx_i)
    for l in range(_L):
        a, b = _tc_proj(h, Wm1[l, :_D], Wm1[l, _D:], bm1[l])
        g = _sc_gather(a, b, idx_i, idx_j)
        msg = _tc_edge(g, Wm2[l], bm2[l], Wa[l].reshape(1, _HID), ba[l])
        agg = _sc_scatter(msg, idx_i)
        h = _tc_node(h, agg, cnt2, Wh1[l, :_D], Wh1[l, _D:], bh1[l],
                     Wh2[l], bh2[l])
    return _tc_final(h, Wo1, bo1, Wo2, bo2, Wo3, bo3)


# speedup vs baseline: 3.4008x; 1.1396x over previous
import functools

import jax
import jax.numpy as jnp
from jax import lax
from jax.experimental import pallas as pl
from jax.experimental.pallas import tpu as pltpu
from jax.experimental.pallas import tpu_sc as plsc

_N = 10000
_E = 160000
_D = 256
_HID = 256
_OUT = 128
_L = 4

_NC = 2
_NS = 16
_NW = _NC * _NS
_LANE = 16

_EW = _E // _NW
_GC = 64
_GT = 80

_SE = _E // _NS
_SCC = 80
_SCT = _SE // _SCC
_FH = _HID // _NC
_NR = 640
_NPAD = _NR * _NS
_NTAIL = _N - 15 * _NR


def _silu(x):
    return x * jax.nn.sigmoid(x)


def _tc_proj(h, w1a, w1b, bm1):
    bn = 2000

    def body(h_ref, wa_ref, wb_ref, b_ref, a_ref, bo_ref):
        hb = h_ref[...]
        a_ref[...] = jnp.dot(hb, wa_ref[...],
                             preferred_element_type=jnp.float32) + b_ref[...]
        bo_ref[...] = jnp.dot(hb, wb_ref[...],
                              preferred_element_type=jnp.float32)

    return pl.pallas_call(
        body,
        grid=(_N // bn,),
        in_specs=[
            pl.BlockSpec((bn, _D), lambda i: (i, 0)),
            pl.BlockSpec((_D, _HID), lambda i: (0, 0)),
            pl.BlockSpec((_D, _HID), lambda i: (0, 0)),
            pl.BlockSpec((1, _HID), lambda i: (0, 0)),
        ],
        out_specs=[pl.BlockSpec((bn, _HID), lambda i: (i, 0))] * 2,
        out_shape=[jax.ShapeDtypeStruct((_N, _HID), jnp.float32)] * 2,
    )(h, w1a, w1b, bm1.reshape(1, _HID))


def _tc_edge(g, wm2, bm2, wa_row, ba):
    be = 1600

    def body(g_ref, w_ref, b_ref, wa_ref, ba_ref, o_ref):
        m = _silu(g_ref[...])
        m2 = _silu(jnp.dot(m, w_ref[...],
                           preferred_element_type=jnp.float32) + b_ref[...])
        logit = jnp.sum(m2 * wa_ref[...], axis=1, keepdims=True) + ba_ref[0, 0]
        o_ref[...] = m2 * jax.nn.sigmoid(logit)

    return pl.pallas_call(
        body,
        grid=(_E // be,),
        in_specs=[
            pl.BlockSpec((be, _HID), lambda i: (i, 0)),
            pl.BlockSpec((_HID, _HID), lambda i: (0, 0)),
            pl.BlockSpec((1, _HID), lambda i: (0, 0)),
            pl.BlockSpec((1, _HID), lambda i: (0, 0)),
            pl.BlockSpec((1, 1), lambda i: (0, 0)),
        ],
        out_specs=pl.BlockSpec((be, _HID), lambda i: (i, 0)),
        out_shape=jax.ShapeDtypeStruct((_E, _HID), jnp.float32),
    )(g, wm2, bm2.reshape(1, _HID), wa_row, ba.reshape(1, 1))


def _tc_node(h, agg, cnt2, wh1h, wh1m, bh1, wh2, bh2):
    bn = 2000

    def body(h_ref, agg_ref, c_ref, w1h_ref, w1m_ref, b1_ref, w2_ref, b2_ref,
             o_ref):
        denom = jnp.maximum(c_ref[:, 0:1], 1.0)
        mean = agg_ref[...] / denom
        z = _silu(
            jnp.dot(h_ref[...], w1h_ref[...],
                    preferred_element_type=jnp.float32)
            + jnp.dot(mean, w1m_ref[...], preferred_element_type=jnp.float32)
            + b1_ref[...])
        o_ref[...] = jnp.dot(z, w2_ref[...],
                             preferred_element_type=jnp.float32) + b2_ref[...]

    return pl.pallas_call(
        body,
        grid=(_N // bn,),
        in_specs=[
            pl.BlockSpec((bn, _D), lambda i: (i, 0)),
            pl.BlockSpec((bn, _HID), lambda i: (i, 0)),
            pl.BlockSpec((bn, _FH), lambda i: (i, 0)),
            pl.BlockSpec((_D, _HID), lambda i: (0, 0)),
            pl.BlockSpec((_HID, _HID), lambda i: (0, 0)),
            pl.BlockSpec((1, _HID), lambda i: (0, 0)),
            pl.BlockSpec((_HID, _HID), lambda i: (0, 0)),
            pl.BlockSpec((1, _HID), lambda i: (0, 0)),
        ],
        out_specs=pl.BlockSpec((bn, _HID), lambda i: (i, 0)),
        out_shape=jax.ShapeDtypeStruct((_N, _HID), jnp.float32),
    )(h, agg, cnt2, wh1h, wh1m, bh1.reshape(1, _HID), wh2,
      bh2.reshape(1, _HID))


def _tc_final(h, wo1, bo1, wo2, bo2, wo3, bo3):
    bn = 1000

    def body(h_ref, w1_ref, b1_ref, w2_ref, b2_ref, w3_ref, b3_ref, o_ref):
        t = _silu(jnp.dot(h_ref[...], w1_ref[...],
                          preferred_element_type=jnp.float32) + b1_ref[...])
        t = jax.nn.relu(jnp.dot(t, w2_ref[...],
                                preferred_element_type=jnp.float32)
                        + b2_ref[...])
        o_ref[...] = jnp.dot(t, w3_ref[...],
                             preferred_element_type=jnp.float32) + b3_ref[...]

    return pl.pallas_call(
        body,
        grid=(_N // bn,),
        in_specs=[
            pl.BlockSpec((bn, _HID), lambda i: (i, 0)),
            pl.BlockSpec((_HID, 1024), lambda i: (0, 0)),
            pl.BlockSpec((1, 1024), lambda i: (0, 0)),
            pl.BlockSpec((1024, 1024), lambda i: (0, 0)),
            pl.BlockSpec((1, 1024), lambda i: (0, 0)),
            pl.BlockSpec((1024, _OUT), lambda i: (0, 0)),
            pl.BlockSpec((1, _OUT), lambda i: (0, 0)),
        ],
        out_specs=pl.BlockSpec((bn, _OUT), lambda i: (i, 0)),
        out_shape=jax.ShapeDtypeStruct((_N, _OUT), jnp.float32),
    )(h, wo1, bo1.reshape(1, 1024), wo2, bo2.reshape(1, 1024), wo3,
      bo3.reshape(1, _OUT))


def _ring(n_chunks, start, process):
    start(0, 0)
    start(1, 1)

    def pair(p, carry):
        for s in (0, 1):
            t = 2 * p + s

            @pl.when(t < n_chunks)
            def _():
                process(t, s)

                @pl.when(t + 2 < n_chunks)
                def _():
                    start(t + 2, s)

        return carry

    lax.fori_loop(0, (n_chunks + 1) // 2, pair, 0)


def _sc_gather(a, b, idx_i, idx_j):
    mesh = plsc.VectorSubcoreMesh(core_axis_name="c", subcore_axis_name="s")

    @functools.partial(
        pl.kernel,
        out_type=jax.ShapeDtypeStruct((_E, _HID), jnp.float32),
        mesh=mesh,
        scratch_types=[
            pltpu.VMEM((_EW,), jnp.int32),
            pltpu.VMEM((_EW,), jnp.int32),
            pltpu.VMEM((2, _GC, _HID), jnp.float32),
            pltpu.VMEM((2, _GC, _HID), jnp.float32),
            pltpu.VMEM((2, _GC, _HID), jnp.float32),
            pltpu.SemaphoreType.DMA,
            pltpu.SemaphoreType.DMA,
            pltpu.SemaphoreType.DMA,
            pltpu.SemaphoreType.DMA,
        ],
    )
    def k(a_hbm, b_hbm, ii_hbm, jj_hbm, g_hbm, ii_v, jj_v, ra_v, rb_v, ob_v,
          gsem0, gsem1, osem0, osem1):
        cid = lax.axis_index("c")
        sid = lax.axis_index("s")
        wid = sid * _NC + cid
        base = wid * _EW
        gsems = (gsem0, gsem1)
        osems = (osem0, osem1)

        pltpu.sync_copy(ii_hbm.at[pl.ds(base, _EW)], ii_v)
        pltpu.sync_copy(jj_hbm.at[pl.ds(base, _EW)], jj_v)

        def loff_of(t):
            return jnp.minimum(t * _GC, _EW - _GC)

        def start(t, s):
            off = loff_of(t)
            pltpu.async_copy(a_hbm.at[ii_v.at[pl.ds(off, _GC)]], ra_v.at[s],
                             gsems[s])
            pltpu.async_copy(b_hbm.at[jj_v.at[pl.ds(off, _GC)]], rb_v.at[s],
                             gsems[s])

        def wait_out(t, s):
            pltpu.make_async_copy(
                ob_v.at[s], g_hbm.at[pl.ds(base + loff_of(t), _GC)],
                osems[s]).wait()

        def process(t, s):
            pltpu.make_async_copy(a_hbm.at[pl.ds(0, _GC)], ra_v.at[s],
                                  gsems[s]).wait()
            pltpu.make_async_copy(b_hbm.at[pl.ds(0, _GC)], rb_v.at[s],
                                  gsems[s]).wait()

            @pl.when(t >= 2)
            def _():
                wait_out(t - 2, s)

            def addrow(e, carry):
                for kk in range(_HID // _LANE):
                    sl = pl.ds(kk * _LANE, _LANE)
                    ob_v[s, e, sl] = ra_v[s, e, sl] + rb_v[s, e, sl]
                return carry

            lax.fori_loop(0, _GC, addrow, 0)
            pltpu.async_copy(ob_v.at[s], g_hbm.at[pl.ds(base + loff_of(t), _GC)],
                             osems[s])

        _ring(_GT, start, process)
        wait_out(_GT - 2, 0)
        wait_out(_GT - 1, 1)

    return k(a, b, idx_i, idx_j)


def _zero_block(z_v, rows):
    def zrow(r, carry):
        zero = jnp.zeros((_LANE,), jnp.float32)
        for kk in range(_FH // _LANE):
            z_v[r, pl.ds(kk * _LANE, _LANE)] = zero
        return carry

    lax.fori_loop(0, rows, zrow, 0)


def _sc_scatter(msg, idx_i):
    mesh = plsc.VectorSubcoreMesh(core_axis_name="c", subcore_axis_name="s")

    @functools.partial(
        pl.kernel,
        out_type=jax.ShapeDtypeStruct((_N, _HID), jnp.float32),
        mesh=mesh,
        scratch_types=[
            pltpu.VMEM((2, _SCC), jnp.int32),
            pltpu.VMEM((2, _SCC, _FH), jnp.float32),
            pltpu.VMEM((_SCC, _FH), jnp.float32),
            pltpu.VMEM_SHARED((_NPAD, _FH), jnp.float32),
            pltpu.SemaphoreType.DMA,
            pltpu.SemaphoreType.DMA,
        ],
    )
    def k(msg_hbm, ii_hbm, agg_hbm, ii_v, mb_v, z_v, acc_sh, sem0, sem1):
        cid = lax.axis_index("c")
        sid = lax.axis_index("s")
        base = sid * _SE
        col0 = cid * _FH
        sems = (sem0, sem1)
        row0 = sid * _NR

        _zero_block(z_v, _SCC)
        for zc in range(_NR // _SCC):
            pltpu.sync_copy(z_v, acc_sh.at[pl.ds(row0 + zc * _SCC, _SCC)])
        plsc.subcore_barrier()

        def start(t, s):
            off = base + t * _SCC
            pltpu.async_copy(ii_hbm.at[pl.ds(off, _SCC)], ii_v.at[s], sems[s])
            pltpu.async_copy(
                msg_hbm.at[pl.ds(off, _SCC), pl.ds(col0, _FH)], mb_v.at[s],
                sems[s])

        def process(t, s):
            pltpu.make_async_copy(ii_hbm.at[pl.ds(0, _SCC)], ii_v.at[s],
                                  sems[s]).wait()
            pltpu.make_async_copy(
                msg_hbm.at[pl.ds(base, _SCC), pl.ds(col0, _FH)],
                mb_v.at[s], sems[s]).wait()
            pltpu.sync_copy(mb_v.at[s], acc_sh.at[ii_v.at[s]], add=True)

        _ring(_SCT, start, process)
        plsc.subcore_barrier()

        @pl.when(sid < _NS - 1)
        def _():
            pltpu.sync_copy(acc_sh.at[pl.ds(row0, _NR)],
                            agg_hbm.at[pl.ds(row0, _NR), pl.ds(col0, _FH)])

        @pl.when(sid == _NS - 1)
        def _():
            pltpu.sync_copy(
                acc_sh.at[pl.ds(row0, _NTAIL)],
                agg_hbm.at[pl.ds(row0, _NTAIL), pl.ds(col0, _FH)])

    return k(msg, idx_i)


def kernel(h, edge_index, Wm1, bm1, Wm2, bm2, Wa, ba, Wh1, bh1, Wh2, bh2,
           Wo1, bo1, Wo2, bo2, Wo3, bo3):
    idx_i = edge_index[0]
    idx_j = edge_index[1]
    cnt2 = _sc_scatter(jnp.ones((_E, _HID), jnp.float32), idx_i)
    for l in range(_L):
        a, b = _tc_proj(h, Wm1[l, :_D], Wm1[l, _D:], bm1[l])
        g = _sc_gather(a, b, idx_i, idx_j)
        msg = _tc_edge(g, Wm2[l], bm2[l], Wa[l].reshape(1, _HID), ba[l])
        agg = _sc_scatter(msg, idx_i)
        h = _tc_node(h, agg, cnt2, Wh1[l, :_D], Wh1[l, _D:], bh1[l],
                     Wh2[l], bh2[l])
    return _tc_final(h, Wo1, bo1, Wo2, bo2, Wo3, bo3)
